# baseline (speedup 1.0000x reference)
import jax
import jax.numpy as jnp
from jax.experimental import pallas as pl
from jax.experimental.pallas import tpu as pltpu

B, N = 128, 8192
RB = 64
NRB = B // RB
CB = 4096
NBLK = N // CB
G = 128
NG = CB // G


def _tc_body(x_ref, m_ref, u_ref, o_ref, carry_ref):
    c = pl.program_id(1)

    @pl.when(c == 0)
    def _():
        carry_ref[...] = jnp.zeros_like(carry_ref)

    masked = x_ref[...] * m_ref[...].astype(jnp.float32)
    u = u_ref[...]
    off = carry_ref[...]
    for g in range(NG):
        s = jnp.dot(masked[:, g * G:(g + 1) * G], u,
                    preferred_element_type=jnp.float32)
        o_ref[:, g * G:(g + 1) * G] = s + off
        off = off + jnp.broadcast_to(s[:, G - 1:G], (RB, G))
    carry_ref[...] = off


def kernel(x, mask):
    u = jnp.triu(jnp.ones((G, G), jnp.float32))
    return pl.pallas_call(
        _tc_body,
        grid=(NRB, NBLK),
        in_specs=[
            pl.BlockSpec((RB, CB), lambda r, c: (r, c)),
            pl.BlockSpec((RB, CB), lambda r, c: (r, c)),
            pl.BlockSpec((G, G), lambda r, c: (0, 0)),
        ],
        out_specs=pl.BlockSpec((RB, CB), lambda r, c: (r, c)),
        out_shape=jax.ShapeDtypeStruct((B, N), jnp.float32),
        scratch_shapes=[pltpu.VMEM((RB, G), jnp.float32)],
    )(x, mask, u)


# speedup vs baseline: 1.1047x; 1.1047x over previous
import jax
import jax.numpy as jnp
from jax.experimental import pallas as pl
from jax.experimental.pallas import tpu as pltpu

B, N = 128, 8192
CB = 4096
NBLK = N // CB
G = 256
NG = CB // G


def _body(x_ref, m_ref, u_ref, o_ref, carry_ref):
    c = pl.program_id(0)

    @pl.when(c == 0)
    def _():
        carry_ref[...] = jnp.zeros_like(carry_ref)

    masked = x_ref[...] * m_ref[...].astype(jnp.float32)
    u = u_ref[...]
    off = carry_ref[...]
    for g in range(NG):
        s = jnp.dot(masked[:, g * G:(g + 1) * G], u,
                    preferred_element_type=jnp.float32)
        o_ref[:, g * G:(g + 1) * G] = s + off
        off = off + jnp.broadcast_to(s[:, G - 1:G], (B, G))
    carry_ref[...] = off


def kernel(x, mask):
    u = jnp.triu(jnp.ones((G, G), jnp.float32))
    return pl.pallas_call(
        _body,
        grid=(NBLK,),
        in_specs=[
            pl.BlockSpec((B, CB), lambda c: (0, c)),
            pl.BlockSpec((B, CB), lambda c: (0, c)),
            pl.BlockSpec((G, G), lambda c: (0, 0)),
        ],
        out_specs=pl.BlockSpec((B, CB), lambda c: (0, c)),
        out_shape=jax.ShapeDtypeStruct((B, N), jnp.float32),
        scratch_shapes=[pltpu.VMEM((B, G), jnp.float32)],
    )(x, mask, u)


# speedup vs baseline: 1.5977x; 1.4463x over previous
import jax
import jax.numpy as jnp
from jax import lax
from jax.experimental import pallas as pl
from jax.experimental.pallas import tpu as pltpu

B, N = 128, 8192
CB = 4096
NBLK = N // CB
G = 256
NG = CB // G


def _body(x_ref, m_ref, o_ref, carry_ref):
    c = pl.program_id(0)

    @pl.when(c == 0)
    def _():
        carry_ref[...] = jnp.zeros_like(carry_ref)

    rows = lax.broadcasted_iota(jnp.int32, (G, G), 0)
    cols = lax.broadcasted_iota(jnp.int32, (G, G), 1)
    u = (rows <= cols).astype(jnp.float32)

    masked = x_ref[...] * m_ref[...].astype(jnp.float32)
    off = carry_ref[...]
    for g in range(NG):
        s = jnp.dot(masked[:, g * G:(g + 1) * G], u,
                    preferred_element_type=jnp.float32)
        o_ref[:, g * G:(g + 1) * G] = s + off
        off = off + jnp.broadcast_to(s[:, G - 1:G], (B, G))
    carry_ref[...] = off


def kernel(x, mask):
    return pl.pallas_call(
        _body,
        grid=(NBLK,),
        in_specs=[
            pl.BlockSpec((B, CB), lambda c: (0, c)),
            pl.BlockSpec((B, CB), lambda c: (0, c)),
        ],
        out_specs=pl.BlockSpec((B, CB), lambda c: (0, c)),
        out_shape=jax.ShapeDtypeStruct((B, N), jnp.float32),
        scratch_shapes=[pltpu.VMEM((B, G), jnp.float32)],
    )(x, mask.astype(jnp.uint8))
